# Initial kernel scaffold; baseline (speedup 1.0000x reference)
import functools

import jax
import jax.numpy as jnp
from jax import lax
from jax.experimental import pallas as pl
from jax.experimental.pallas import tpu as pltpu
from jax.experimental.pallas import tpu_sc as plsc

IN_DIM = 1024
HID = 256
E = 16
T = 4096
BT = 256
NBLK = T // BT
NITEM = NBLK + E - 1


def _router_body(x_ref, wrt_ref, br_ref, out_ref):
    logits = jnp.dot(x_ref[...], wrt_ref[...],
                     preferred_element_type=jnp.float32) + br_ref[...]
    mx = jnp.max(logits, axis=1, keepdims=True)
    idx = lax.broadcasted_iota(jnp.int32, logits.shape, 1)
    out_ref[...] = jnp.min(jnp.where(logits >= mx, idx, E), axis=1,
                           keepdims=True)


def _router(xf, WrT, br2):
    bt = 512
    return pl.pallas_call(
        _router_body,
        grid=(T // bt,),
        in_specs=[
            pl.BlockSpec((bt, IN_DIM), lambda i: (i, 0)),
            pl.BlockSpec((IN_DIM, E), lambda i: (0, 0)),
            pl.BlockSpec((1, E), lambda i: (0, 0)),
        ],
        out_specs=pl.BlockSpec((bt, 1), lambda i: (i, 0)),
        out_shape=jax.ShapeDtypeStruct((T, 1), jnp.int32),
    )(xf, WrT, br2)


def _make_row_gather():
    info = plsc.get_sparse_core_info()
    nw = info.num_cores * info.num_subcores
    rows_per_w = T // nw
    chunk = 64
    mesh = plsc.VectorSubcoreMesh(core_axis_name="c", subcore_axis_name="s")

    @functools.partial(
        pl.kernel,
        out_type=jax.ShapeDtypeStruct((T, IN_DIM), jnp.float32),
        mesh=mesh,
        scratch_types=[
            pltpu.VMEM((rows_per_w,), jnp.int32),
            pltpu.VMEM((chunk, IN_DIM), jnp.float32),
            pltpu.SemaphoreType.DMA,
        ],
    )
    def gather_rows(src_hbm, idx_hbm, out_hbm, idx_v, rows_v, sem):
        wid = lax.axis_index("s") * info.num_cores + lax.axis_index("c")
        base = wid * rows_per_w
        pltpu.sync_copy(idx_hbm.at[pl.ds(base, rows_per_w)], idx_v)
        for k in range(rows_per_w // chunk):
            pltpu.async_copy(
                src_hbm.at[idx_v.at[pl.ds(k * chunk, chunk)]], rows_v, sem
            ).wait()
            pltpu.sync_copy(rows_v, out_hbm.at[pl.ds(base + k * chunk, chunk)])

    return gather_rows


_gather_rows = _make_row_gather()


def _mlp_body(sched_ref, x_ref, eid_ref, w1_ref, b1_ref, w2_ref, b2_ref,
              out_ref):
    i = pl.program_id(0)
    e = sched_ref[1, i]
    valid = sched_ref[2, i]
    first = sched_ref[3, i]

    h = jnp.dot(x_ref[...], w1_ref[0], preferred_element_type=jnp.float32)
    h = h + b1_ref[pl.ds(e, 1), :]
    h = 0.5 * h * (1.0 + lax.erf(h * 0.7071067811865476))
    o = jnp.dot(h, w2_ref[0], preferred_element_type=jnp.float32)
    o = o + b2_ref[pl.ds(e, 1), :]
    mask = (eid_ref[...] == e) & (valid != 0)
    contrib = jnp.where(mask, o, 0.0)

    @pl.when(first != 0)
    def _():
        out_ref[...] = x_ref[...] + contrib

    @pl.when(first == 0)
    def _():
        out_ref[...] = out_ref[...] + contrib


def _grouped_mlp(sched, x_sorted, eid_sorted, W1, b1, W2, b2):
    grid_spec = pltpu.PrefetchScalarGridSpec(
        num_scalar_prefetch=1,
        grid=(NITEM,),
        in_specs=[
            pl.BlockSpec((BT, IN_DIM), lambda i, s: (s[0, i], 0)),
            pl.BlockSpec((BT, 1), lambda i, s: (s[0, i], 0)),
            pl.BlockSpec((1, IN_DIM, HID), lambda i, s: (s[1, i], 0, 0)),
            pl.BlockSpec((E, HID), lambda i, s: (0, 0)),
            pl.BlockSpec((1, HID, IN_DIM), lambda i, s: (s[1, i], 0, 0)),
            pl.BlockSpec((E, IN_DIM), lambda i, s: (0, 0)),
        ],
        out_specs=pl.BlockSpec((BT, IN_DIM), lambda i, s: (s[0, i], 0)),
    )
    return pl.pallas_call(
        _mlp_body,
        grid_spec=grid_spec,
        out_shape=jax.ShapeDtypeStruct((T, IN_DIM), jnp.float32),
    )(sched, x_sorted, eid_sorted, W1, b1, W2, b2)


def kernel(x, Wr, br, W1, b1, W2, b2):
    token_shape = x.shape[:-1]
    xf = x.reshape(T, IN_DIM)

    eid2 = _router(xf, Wr.T, br.reshape(1, E))
    eid = eid2[:, 0]

    perm = jnp.argsort(eid, stable=True).astype(jnp.int32)
    pos = jnp.zeros((T,), jnp.int32).at[perm].set(
        jnp.arange(T, dtype=jnp.int32))
    eid_sorted = eid2[perm]

    counts = jnp.bincount(eid, length=E)
    starts = jnp.concatenate([jnp.zeros((1,), counts.dtype),
                              jnp.cumsum(counts)[:-1]])
    ends = starts + counts
    blo = jnp.arange(NBLK)[:, None] * BT
    ov = ((starts[None, :] < blo + BT) & (ends[None, :] > blo)
          & (counts[None, :] > 0))
    (flat_idx,) = jnp.nonzero(ov.reshape(-1), size=NITEM,
                              fill_value=NBLK * E - 1)
    nvalid = jnp.sum(ov)
    b_i = (flat_idx // E).astype(jnp.int32)
    e_i = (flat_idx % E).astype(jnp.int32)
    valid = (jnp.arange(NITEM) < nvalid).astype(jnp.int32)
    is_first = jnp.concatenate(
        [jnp.ones((1,), jnp.int32),
         (b_i[1:] != b_i[:-1]).astype(jnp.int32)])
    sched = jnp.stack([b_i, e_i, valid, is_first])

    x_sorted = _gather_rows(xf, perm)
    y_sorted = _grouped_mlp(sched, x_sorted, eid_sorted, W1, b1, W2, b2)
    y = _gather_rows(y_sorted, pos)

    return y.reshape(*token_shape, IN_DIM)


# speedup vs baseline: 4.1130x; 4.1130x over previous
import functools

import jax
import jax.numpy as jnp
from jax import lax
from jax.experimental import pallas as pl
from jax.experimental.pallas import tpu as pltpu
from jax.experimental.pallas import tpu_sc as plsc

IN_DIM = 1024
HID = 256
E = 16
T = 4096
BT = 256
NBLK = T // BT
NITEM = NBLK + E - 1


def _router_body(x_ref, wrt_ref, br_ref, out_ref):
    logits = jnp.dot(x_ref[...], wrt_ref[...],
                     preferred_element_type=jnp.float32) + br_ref[...]
    mx = jnp.max(logits, axis=1, keepdims=True)
    idx = lax.broadcasted_iota(jnp.int32, logits.shape, 1)
    out_ref[...] = jnp.min(jnp.where(logits >= mx, idx, E), axis=1,
                           keepdims=True)


def _router(xf, WrT, br2):
    bt = 512
    return pl.pallas_call(
        _router_body,
        grid=(T // bt,),
        in_specs=[
            pl.BlockSpec((bt, IN_DIM), lambda i: (i, 0)),
            pl.BlockSpec((IN_DIM, E), lambda i: (0, 0)),
            pl.BlockSpec((1, E), lambda i: (0, 0)),
        ],
        out_specs=pl.BlockSpec((bt, 1), lambda i: (i, 0)),
        out_shape=jax.ShapeDtypeStruct((T, 1), jnp.int32),
    )(xf, WrT, br2)


@functools.cache
def _make_row_gather():
    info = plsc.get_sparse_core_info()
    nw = info.num_cores * info.num_subcores
    rows_per_w = T // nw
    chunk = 64
    mesh = plsc.VectorSubcoreMesh(core_axis_name="c", subcore_axis_name="s")

    @functools.partial(
        pl.kernel,
        out_type=jax.ShapeDtypeStruct((T, IN_DIM), jnp.float32),
        mesh=mesh,
        scratch_types=[
            pltpu.VMEM((rows_per_w,), jnp.int32),
            pltpu.VMEM((chunk, IN_DIM), jnp.float32),
            pltpu.SemaphoreType.DMA,
        ],
    )
    def gather_rows(src_hbm, idx_hbm, out_hbm, idx_v, rows_v, sem):
        wid = lax.axis_index("s") * info.num_cores + lax.axis_index("c")
        base = wid * rows_per_w
        pltpu.sync_copy(idx_hbm.at[pl.ds(base, rows_per_w)], idx_v)
        for k in range(rows_per_w // chunk):
            pltpu.async_copy(
                src_hbm.at[idx_v.at[pl.ds(k * chunk, chunk)]], rows_v, sem
            ).wait()
            pltpu.sync_copy(rows_v, out_hbm.at[pl.ds(base + k * chunk, chunk)])

    return gather_rows


def _gather_rows(src, idx):
    return _make_row_gather()(src, idx)


def _mlp_body(sched_ref, x_ref, eid_ref, w1_ref, b1_ref, w2_ref, b2_ref,
              out_ref):
    i = pl.program_id(0)
    e = sched_ref[1, i]
    valid = sched_ref[2, i]
    first = sched_ref[3, i]

    h = jnp.dot(x_ref[...], w1_ref[0], preferred_element_type=jnp.float32)
    h = h + b1_ref[pl.ds(e, 1), :]
    h = 0.5 * h * (1.0 + lax.erf(h * 0.7071067811865476))
    o = jnp.dot(h, w2_ref[0], preferred_element_type=jnp.float32)
    o = o + b2_ref[pl.ds(e, 1), :]
    mask = (eid_ref[...] == e) & (valid != 0)
    contrib = jnp.where(mask, o, 0.0)

    @pl.when(first != 0)
    def _():
        out_ref[...] = x_ref[...] + contrib

    @pl.when(first == 0)
    def _():
        out_ref[...] = out_ref[...] + contrib


def _grouped_mlp(sched, x_sorted, eid_sorted, W1, b1, W2, b2):
    grid_spec = pltpu.PrefetchScalarGridSpec(
        num_scalar_prefetch=1,
        grid=(NITEM,),
        in_specs=[
            pl.BlockSpec((BT, IN_DIM), lambda i, s: (s[0, i], 0)),
            pl.BlockSpec((BT, 1), lambda i, s: (s[0, i], 0)),
            pl.BlockSpec((1, IN_DIM, HID), lambda i, s: (s[1, i], 0, 0)),
            pl.BlockSpec((E, HID), lambda i, s: (0, 0)),
            pl.BlockSpec((1, HID, IN_DIM), lambda i, s: (s[1, i], 0, 0)),
            pl.BlockSpec((E, IN_DIM), lambda i, s: (0, 0)),
        ],
        out_specs=pl.BlockSpec((BT, IN_DIM), lambda i, s: (s[0, i], 0)),
    )
    return pl.pallas_call(
        _mlp_body,
        grid_spec=grid_spec,
        out_shape=jax.ShapeDtypeStruct((T, IN_DIM), jnp.float32),
    )(sched, x_sorted, eid_sorted, W1, b1, W2, b2)


def kernel(x, Wr, br, W1, b1, W2, b2):
    token_shape = x.shape[:-1]
    xf = x.reshape(T, IN_DIM)

    eid2 = _router(xf, Wr.T, br.reshape(1, E))
    eid = eid2[:, 0]

    perm = jnp.argsort(eid, stable=True).astype(jnp.int32)
    pos = jnp.zeros((T,), jnp.int32).at[perm].set(
        jnp.arange(T, dtype=jnp.int32))
    eid_sorted = eid2[perm]

    counts = jnp.bincount(eid, length=E)
    starts = jnp.concatenate([jnp.zeros((1,), counts.dtype),
                              jnp.cumsum(counts)[:-1]])
    ends = starts + counts
    blo = jnp.arange(NBLK)[:, None] * BT
    ov = ((starts[None, :] < blo + BT) & (ends[None, :] > blo)
          & (counts[None, :] > 0))
    (flat_idx,) = jnp.nonzero(ov.reshape(-1), size=NITEM,
                              fill_value=NBLK * E - 1)
    nvalid = jnp.sum(ov)
    b_i = (flat_idx // E).astype(jnp.int32)
    e_i = (flat_idx % E).astype(jnp.int32)
    valid = (jnp.arange(NITEM) < nvalid).astype(jnp.int32)
    is_first = jnp.concatenate(
        [jnp.ones((1,), jnp.int32),
         (b_i[1:] != b_i[:-1]).astype(jnp.int32)])
    sched = jnp.stack([b_i, e_i, valid, is_first])

    x_sorted = _gather_rows(xf, perm)
    y_sorted = _grouped_mlp(sched, x_sorted, eid_sorted, W1, b1, W2, b2)
    y = _gather_rows(y_sorted, pos)

    return y.reshape(*token_shape, IN_DIM)
